# Initial kernel scaffold; baseline (speedup 1.0000x reference)
import jax
import jax.numpy as jnp
from jax.experimental import pallas as pl


def kernel(atomic_contributions, atoms, graph_indexes, scale, shift, atom_refs):
    raise NotImplementedError("write your pallas kernel here")


# speedup vs baseline: 345.0098x; 345.0098x over previous
import functools

import jax
import jax.numpy as jnp
from jax import lax
from jax.experimental import pallas as pl
from jax.experimental.pallas import tpu as pltpu
from jax.experimental.pallas import tpu_sc as plsc

N = 6_400_000
NUM_GRAPHS = 100_000
ACC = 100_096
NW = 32
CHUNK = N // NW
C = 10_000
NSUB = CHUNK // C
S = C // 16
UNROLL = 5


def _sc_body(contrib_hbm, packed_hbm, refs_hbm, sv_hbm, sh_hbm,
             out_hbm, acc, cbuf, pbuf, refsv, svv, shv, sem):
    wid = lax.axis_index("s") * 2 + lax.axis_index("c")
    base = wid * CHUNK

    pltpu.sync_copy(refs_hbm, refsv)
    pltpu.sync_copy(sv_hbm, svv)
    pltpu.sync_copy(sh_hbm, shv)
    svec = svv[...]
    shvec = shv[...]
    lane = lax.broadcasted_iota(jnp.int32, (16,), 0)
    zf = jnp.zeros((16,), jnp.float32)

    def zero16(t, _):
        for u in range(16):
            acc[pl.ds(256 * t + 16 * u, 16)] = zf
        return 0
    lax.fori_loop(0, ACC // 256, zero16, 0)

    def sub_chunk(j, _):
        off = pl.multiple_of(base + j * C, 8)
        d1 = pltpu.async_copy(contrib_hbm.at[pl.ds(off, C)], cbuf, sem)
        d2 = pltpu.async_copy(packed_hbm.at[pl.ds(off, C)], pbuf, sem)
        d1.wait()
        d2.wait()

        def one_step(ix, run, gprev):
            p = plsc.load_gather(pbuf, [ix])
            c = plsc.load_gather(cbuf, [ix])
            gv = p >> 11
            rix = (p & 2047) + lane
            r = plsc.load_gather(refsv, [rix])
            v = c * svec + shvec + r
            m = gv != gprev
            plsc.addupdate_scatter(acc, [gprev], run, mask=m)
            run = jnp.where(m, 0.0, run) + v
            return ix + 1, run, gv

        def step(i, carry):
            ix, run, gprev = carry
            for _ in range(UNROLL):
                ix, run, gprev = one_step(ix, run, gprev)
            return ix, run, gprev

        ix0 = lane * S
        g0 = plsc.load_gather(pbuf, [ix0]) >> 11
        _, run, gprev = lax.fori_loop(0, S // UNROLL, step, (ix0, zf, g0))
        for l in range(16):
            plsc.addupdate_scatter(acc, [gprev], run, mask=lane == l)
        return 0
    lax.fori_loop(0, NSUB, sub_chunk, 0)

    row = wid * ACC
    ds = [pltpu.async_copy(acc.at[pl.ds(k * (ACC // 16), ACC // 16)],
                           out_hbm.at[pl.ds(row + k * (ACC // 16), ACC // 16)],
                           sem)
          for k in range(16)]
    for d in ds:
        d.wait()


def _tc_sum_body(x_ref, o_ref):
    o_ref[...] = jnp.sum(x_ref[...], axis=0)


def kernel(atomic_contributions, atoms, graph_indexes, scale, shift, atom_refs):
    contrib = atomic_contributions.reshape(-1).astype(jnp.float32)
    packed = (graph_indexes.reshape(-1).astype(jnp.int32) << 11) | (
        atoms.reshape(-1).astype(jnp.int32) << 4)
    refs_rep = jnp.tile(atom_refs.reshape(-1, 1).astype(jnp.float32),
                        (1, 16)).reshape(-1)
    sv = jnp.full((16,), scale, jnp.float32)
    sh = jnp.full((16,), shift, jnp.float32)

    mesh = plsc.VectorSubcoreMesh(core_axis_name="c", subcore_axis_name="s")
    sc = functools.partial(
        pl.kernel,
        mesh=mesh,
        compiler_params=pltpu.CompilerParams(needs_layout_passes=False),
        out_type=jax.ShapeDtypeStruct((NW * ACC,), jnp.float32),
        scratch_types=[
            pltpu.VMEM((ACC,), jnp.float32),
            pltpu.VMEM((C,), jnp.float32),
            pltpu.VMEM((C,), jnp.int32),
            pltpu.VMEM((1600,), jnp.float32),
            pltpu.VMEM((16,), jnp.float32),
            pltpu.VMEM((16,), jnp.float32),
            pltpu.SemaphoreType.DMA,
        ],
    )(_sc_body)
    partial_rows = sc(contrib, packed, refs_rep, sv, sh)

    summed = pl.pallas_call(
        _tc_sum_body,
        out_shape=jax.ShapeDtypeStruct((ACC // 128, 128), jnp.float32),
    )(partial_rows.reshape(NW, ACC // 128, 128))
    return summed.reshape(-1)[:NUM_GRAPHS].reshape(NUM_GRAPHS, 1)
